# baseline (speedup 1.0000x reference)
import functools

import jax
import jax.numpy as jnp
from jax import lax
from jax.experimental import pallas as pl
from jax.experimental.pallas import tpu as pltpu
from jax.experimental.pallas import tpu_sc as plsc

_CHUNK = 128


def _sc_gather(emb, idx3, *, nw, chunks):
    nrows = nw * chunks * _CHUNK
    embed_dim = emb.shape[1]
    mesh = plsc.VectorSubcoreMesh(core_axis_name="c", subcore_axis_name="s")
    num_cores = mesh.num_cores

    @functools.partial(
        pl.kernel,
        out_type=jax.ShapeDtypeStruct((nrows, embed_dim), jnp.float32),
        mesh=mesh,
        scratch_types=[
            pltpu.VMEM((chunks, _CHUNK), jnp.int32),
            pltpu.VMEM((2, _CHUNK, embed_dim), jnp.float32),
            pltpu.SemaphoreType.DMA,
            pltpu.SemaphoreType.DMA,
        ],
    )
    def gather_kernel(emb_hbm, idx_hbm, out_hbm, idx_v, rows_v, sem0, sem1):
        wid = lax.axis_index("s") * num_cores + lax.axis_index("c")
        base = wid * (chunks * _CHUNK)
        pltpu.sync_copy(idx_hbm.at[wid], idx_v)
        sems = (sem0, sem1)

        def start(g):
            return pltpu.async_copy(
                emb_hbm.at[idx_v.at[g]], rows_v.at[g % 2], sems[g % 2]
            )

        pending = start(0)
        for g in range(chunks):
            nxt = start(g + 1) if g + 1 < chunks else None
            pending.wait()
            pltpu.sync_copy(
                rows_v.at[g % 2], out_hbm.at[pl.ds(base + g * _CHUNK, _CHUNK)]
            )
            pending = nxt

    return gather_kernel(emb, idx3)


def _fuse_weights(W1, W2, b1, b2):
    def body(w1_ref, w2_ref, b1_ref, b2_ref, wf_ref, bf_ref):
        w2 = w2_ref[...]
        wf_ref[...] = jnp.dot(
            w1_ref[...], w2,
            preferred_element_type=jnp.float32,
            precision=lax.Precision.HIGHEST,
        )
        bf_ref[...] = (
            jnp.dot(
                b1_ref[...], w2,
                preferred_element_type=jnp.float32,
                precision=lax.Precision.HIGHEST,
            )
            + b2_ref[...]
        )

    embed_dim, hidden = W1.shape
    num_out = W2.shape[1]
    return pl.pallas_call(
        body,
        out_shape=(
            jax.ShapeDtypeStruct((embed_dim, num_out), jnp.float32),
            jax.ShapeDtypeStruct((1, num_out), jnp.float32),
        ),
    )(W1, W2, b1.reshape(1, hidden), b2.reshape(1, num_out))


def _mlp(gathered, wf, bf, *, block_rows):
    nrows, embed_dim = gathered.shape
    num_out = wf.shape[1]

    def body(x_ref, wf_ref, bf_ref, o_ref):
        o_ref[...] = (
            jnp.dot(x_ref[...], wf_ref[...], preferred_element_type=jnp.float32)
            + bf_ref[...]
        )

    return pl.pallas_call(
        body,
        grid=(nrows // block_rows,),
        in_specs=[
            pl.BlockSpec((block_rows, embed_dim), lambda i: (i, 0)),
            pl.BlockSpec((embed_dim, num_out), lambda i: (0, 0)),
            pl.BlockSpec((1, num_out), lambda i: (0, 0)),
        ],
        out_specs=pl.BlockSpec((block_rows, num_out), lambda i: (i, 0)),
        out_shape=jax.ShapeDtypeStruct((nrows, num_out), jnp.float32),
    )(gathered, wf, bf)


def kernel(input_notes, emb, W1, b1, W2, b2):
    batch, hist = input_notes.shape
    nrows = batch * hist
    info = plsc.get_sparse_core_info()
    nw = info.num_cores * info.num_subcores
    chunks = nrows // (nw * _CHUNK)
    idx3 = input_notes.reshape(nw, chunks, _CHUNK).astype(jnp.int32)

    gathered = _sc_gather(emb, idx3, nw=nw, chunks=chunks)
    wf, bf = _fuse_weights(W1, W2, b1, b2)
    out = _mlp(gathered, wf, bf, block_rows=1024)
    return out.reshape(batch, hist, W2.shape[1])
